# Initial kernel scaffold; baseline (speedup 1.0000x reference)
import functools

import jax
import jax.numpy as jnp
from jax.experimental import pallas as pl

_K = 128
_TEMP = 0.7
_TB = 128


def _norm_body(x_ref, o_ref):
    x = x_ref[...]
    n = jnp.sqrt(jnp.sum(x * x, axis=1, keepdims=True))
    o_ref[...] = x / jnp.maximum(n, 1e-12)


def _route_body(base_ref, wq_ref, bq_ref, an_ref, o_ref):
    q = jax.lax.dot_general(
        base_ref[...], wq_ref[...], (((1,), (1,)), ((), ())),
        preferred_element_type=jnp.float32) + bq_ref[...]
    n = jnp.sqrt(jnp.sum(q * q, axis=1, keepdims=True))
    q = q / jnp.maximum(n, 1e-12)
    s = jax.lax.dot_general(
        q, an_ref[...], (((1,), (1,)), ((), ())),
        preferred_element_type=jnp.float32)
    key = jax.lax.bitcast_convert_type(s, jnp.int32)
    key = key ^ (jax.lax.shift_right_arithmetic(key, 31) & jnp.int32(0x7FFFFFFF))

    def bit_step(i, t):
        cand = t + jax.lax.shift_left(jnp.int32(1), jnp.int32(30) - i)
        cnt = jnp.sum((key >= cand).astype(jnp.int32), axis=1, keepdims=True)
        return jnp.where(cnt >= _K, cand, t)

    t0 = jnp.full((s.shape[0], 1), jnp.int32(-2147483647) - 1)
    t = jax.lax.fori_loop(0, 31, bit_step, t0)

    mask = key >= t
    m = jnp.max(s, axis=1, keepdims=True)
    e = jnp.where(mask, jnp.exp((s - m) / _TEMP), 0.0)
    w = e / jnp.sum(e, axis=1, keepdims=True)
    o_ref[...] = jnp.dot(w, an_ref[...], preferred_element_type=jnp.float32)


def _build(B, D, A, interpret=False):
    norm = pl.pallas_call(
        _norm_body,
        grid=(A // 1024,),
        in_specs=[pl.BlockSpec((1024, D), lambda i: (i, 0))],
        out_specs=pl.BlockSpec((1024, D), lambda i: (i, 0)),
        out_shape=jax.ShapeDtypeStruct((A, D), jnp.float32),
        interpret=interpret,
    )
    route = pl.pallas_call(
        _route_body,
        grid=(B // _TB,),
        in_specs=[
            pl.BlockSpec((_TB, D), lambda i: (i, 0)),
            pl.BlockSpec((D, D), lambda i: (0, 0)),
            pl.BlockSpec((1, D), lambda i: (0, 0)),
            pl.BlockSpec((A, D), lambda i: (0, 0)),
        ],
        out_specs=pl.BlockSpec((_TB, D), lambda i: (i, 0)),
        out_shape=jax.ShapeDtypeStruct((B, D), jnp.float32),
        interpret=interpret,
    )
    return norm, route


def kernel(base_raw, Wq, bq, atom_bank):
    B, D = base_raw.shape
    A = atom_bank.shape[0]
    norm, route = _build(B, D, A)
    an = norm(atom_bank)
    return route(base_raw, Wq, bq.reshape(1, D), an)


# speedup vs baseline: 15.5802x; 15.5802x over previous
import jax
import jax.numpy as jnp
from jax.experimental import pallas as pl

_K = 128
_TEMP = 0.7
_TB = 128


def _norm_body(x_ref, an_ref):
    x = x_ref[...]
    n = jnp.sqrt(jnp.sum(x * x, axis=1, keepdims=True))
    an_ref[...] = (x / jnp.maximum(n, 1e-12)).astype(jnp.bfloat16)


def _route_body(base_ref, wq_ref, bq_ref, atom_ref, an_ref, o_ref):
    q = jax.lax.dot_general(
        base_ref[...].astype(jnp.bfloat16), wq_ref[...].astype(jnp.bfloat16),
        (((1,), (1,)), ((), ())),
        preferred_element_type=jnp.float32) + bq_ref[...]
    n = jnp.sqrt(jnp.sum(q * q, axis=1, keepdims=True))
    q = q / jnp.maximum(n, 1e-12)
    s = jax.lax.dot_general(
        q.astype(jnp.bfloat16), an_ref[...],
        (((1,), (1,)), ((), ())),
        preferred_element_type=jnp.float32)
    key = jax.lax.bitcast_convert_type(s, jnp.int32)
    key = key ^ (jax.lax.shift_right_arithmetic(key, 31) & jnp.int32(0x7FFFFFFF))

    def bit_step(i, t):
        cand = t + jax.lax.shift_left(jnp.int32(1), jnp.int32(31) - i)
        cnt = jnp.sum((key >= cand).astype(jnp.int32), axis=1, keepdims=True)
        return jnp.where(cnt >= _K, cand, t)

    t0 = jnp.full((s.shape[0], 1), jnp.int32(-2147483647) - 1)
    t = jax.lax.fori_loop(0, 32, bit_step, t0)

    mask = key >= t
    m = jnp.max(s, axis=1, keepdims=True)
    e = jnp.where(mask, jnp.exp((s - m) / _TEMP), 0.0)
    w = e / jnp.sum(e, axis=1, keepdims=True)
    o_ref[...] = jax.lax.dot_general(
        w.astype(jnp.bfloat16), atom_ref[...].astype(jnp.bfloat16),
        (((1,), (0,)), ((), ())),
        preferred_element_type=jnp.float32)


def _build(B, D, A, interpret=False):
    norm = pl.pallas_call(
        _norm_body,
        grid=(A // 1024,),
        in_specs=[pl.BlockSpec((1024, D), lambda i: (i, 0))],
        out_specs=pl.BlockSpec((1024, D), lambda i: (i, 0)),
        out_shape=jax.ShapeDtypeStruct((A, D), jnp.bfloat16),
        interpret=interpret,
    )
    route = pl.pallas_call(
        _route_body,
        grid=(B // _TB,),
        in_specs=[
            pl.BlockSpec((_TB, D), lambda i: (i, 0)),
            pl.BlockSpec((D, D), lambda i: (0, 0)),
            pl.BlockSpec((1, D), lambda i: (0, 0)),
            pl.BlockSpec((A, D), lambda i: (0, 0)),
            pl.BlockSpec((A, D), lambda i: (0, 0)),
        ],
        out_specs=pl.BlockSpec((_TB, D), lambda i: (i, 0)),
        out_shape=jax.ShapeDtypeStruct((B, D), jnp.float32),
        interpret=interpret,
    )
    return norm, route


def kernel(base_raw, Wq, bq, atom_bank):
    B, D = base_raw.shape
    A = atom_bank.shape[0]
    norm, route = _build(B, D, A)
    an = norm(atom_bank)
    return route(base_raw, Wq, bq.reshape(1, D), atom_bank, an)
